# initial kernel scaffold (unmeasured)
import jax
import jax.numpy as jnp
from jax import lax
from jax.experimental import pallas as pl
from jax.experimental.pallas import tpu as pltpu

N_DEV = 8
L_CORR = 256


def kernel(x, A, B, C):
    Bb, L, D = x.shape
    N = A.shape[1]

    def body(x_ref, a_ref, b_ref, c_ref, out_ref,
             carry_ref, hend_ref, bT_ref, cT_ref, send_sem, recv_sem):
        my = lax.axis_index("i")

        @pl.when(my == 0)
        def _():
            carry_ref[...] = jnp.zeros_like(carry_ref)

        dAT = jnp.exp(a_ref[...].T)

        for b in range(Bb):
            bT_ref[b] = b_ref[b].T
            cT_ref[b] = c_ref[b].T

        def step(t, h):
            x_t = x_ref[:, pl.ds(t, 1), :][:, 0, :]
            b_t = bT_ref[:, :, pl.ds(t, 1)]
            c_t = cT_ref[:, :, pl.ds(t, 1)]
            h = h * dAT[None] + b_t * x_t[:, None, :]
            y_t = jnp.sum(h * c_t, axis=1)
            out_ref[:, pl.ds(t, 1), :] = y_t[:, None, :]
            return h

        h_end = lax.fori_loop(
            0, L, step, jnp.zeros((Bb, N, D), jnp.float32))
        hend_ref[...] = h_end

        rdma = pltpu.make_async_remote_copy(
            src_ref=hend_ref,
            dst_ref=carry_ref,
            send_sem=send_sem,
            recv_sem=recv_sem,
            device_id=((my + 1) % N_DEV,),
            device_id_type=pl.DeviceIdType.MESH,
        )

        @pl.when(my < N_DEV - 1)
        def _():
            rdma.start()

        @pl.when(my > 0)
        def _():
            rdma.wait_recv()

        def corr_step(t, p):
            c_t = cT_ref[:, :, pl.ds(t, 1)]
            y_t = out_ref[:, pl.ds(t, 1), :][:, 0, :]
            y_t = y_t + jnp.sum(p * c_t, axis=1)
            out_ref[:, pl.ds(t, 1), :] = y_t[:, None, :]
            return p * dAT[None]

        lax.fori_loop(0, L_CORR, corr_step, dAT[None] * carry_ref[...])

        @pl.when(my < N_DEV - 1)
        def _():
            rdma.wait_send()

    return pl.pallas_call(
        body,
        out_shape=jax.ShapeDtypeStruct((Bb, L, D), jnp.float32),
        in_specs=[pl.BlockSpec(memory_space=pltpu.VMEM)] * 4,
        out_specs=pl.BlockSpec(memory_space=pltpu.VMEM),
        scratch_shapes=[
            pltpu.VMEM((Bb, N, D), jnp.float32),
            pltpu.VMEM((Bb, N, D), jnp.float32),
            pltpu.VMEM((Bb, N, L), jnp.float32),
            pltpu.VMEM((Bb, N, L), jnp.float32),
            pltpu.SemaphoreType.DMA,
            pltpu.SemaphoreType.DMA,
        ],
        compiler_params=pltpu.CompilerParams(collective_id=0),
    )(x, A, B, C)


# baseline (device time: 397415 ns/iter reference)
import jax
import jax.numpy as jnp
from jax import lax
from jax.experimental import pallas as pl
from jax.experimental.pallas import tpu as pltpu

N_DEV = 8
L_CORR = 256


def kernel(x, A, B, C):
    Bb, L, D = x.shape
    N = A.shape[1]

    def body(x_ref, a_ref, b_ref, c_ref, out_ref,
             carry_ref, hend_ref, send_sem, recv_sem):
        my = lax.axis_index("i")

        @pl.when(my == 0)
        def _():
            carry_ref[...] = jnp.zeros_like(carry_ref)

        dAT = jnp.exp(a_ref[...].T)

        def step(t, h):
            x_t = x_ref[:, pl.ds(t, 1), :][:, 0, :]
            b_t = jnp.swapaxes(b_ref[:, pl.ds(t, 1), :], 1, 2)
            c_t = jnp.swapaxes(c_ref[:, pl.ds(t, 1), :], 1, 2)
            h = h * dAT[None] + b_t * x_t[:, None, :]
            y_t = jnp.sum(h * c_t, axis=1)
            out_ref[:, pl.ds(t, 1), :] = y_t[:, None, :]
            return h

        h_end = lax.fori_loop(
            0, L, step, jnp.zeros((Bb, N, D), jnp.float32))
        hend_ref[...] = h_end

        rdma = pltpu.make_async_remote_copy(
            src_ref=hend_ref,
            dst_ref=carry_ref,
            send_sem=send_sem,
            recv_sem=recv_sem,
            device_id=((my + 1) % N_DEV,),
            device_id_type=pl.DeviceIdType.MESH,
        )

        @pl.when(my < N_DEV - 1)
        def _():
            rdma.start()

        @pl.when(my > 0)
        def _():
            rdma.wait_recv()

        def corr_step(t, p):
            c_t = jnp.swapaxes(c_ref[:, pl.ds(t, 1), :], 1, 2)
            y_t = out_ref[:, pl.ds(t, 1), :][:, 0, :]
            y_t = y_t + jnp.sum(p * c_t, axis=1)
            out_ref[:, pl.ds(t, 1), :] = y_t[:, None, :]
            return p * dAT[None]

        lax.fori_loop(0, L_CORR, corr_step, dAT[None] * carry_ref[...])

        @pl.when(my < N_DEV - 1)
        def _():
            rdma.wait_send()

    return pl.pallas_call(
        body,
        out_shape=jax.ShapeDtypeStruct((Bb, L, D), jnp.float32),
        in_specs=[pl.BlockSpec(memory_space=pltpu.VMEM)] * 4,
        out_specs=pl.BlockSpec(memory_space=pltpu.VMEM),
        scratch_shapes=[
            pltpu.VMEM((Bb, N, D), jnp.float32),
            pltpu.VMEM((Bb, N, D), jnp.float32),
            pltpu.SemaphoreType.DMA,
            pltpu.SemaphoreType.DMA,
        ],
    )(x, A, B, C)
